# baseline (speedup 1.0000x reference)
import functools

import jax
import jax.numpy as jnp
from jax import lax
from jax.experimental import pallas as pl
from jax.experimental.pallas import tpu as pltpu
from jax.experimental.pallas import tpu_sc as plsc

D = 64
NB = 16384
M = 8
K = 8
NC = 2
NS = 16
NW = NC * NS
BPW = NB // NW
CHUNK = 32
NCHUNKS = BPW // CHUNK
GROUPS = CHUNK // 16
XPC = CHUNK * M
XSTREAMS = XPC // 128

_mesh = plsc.VectorSubcoreMesh(core_axis_name="c", subcore_axis_name="s")


@functools.partial(
    pl.kernel,
    out_type=(
        jax.ShapeDtypeStruct((NB * M,), jnp.float32),
        jax.ShapeDtypeStruct((NB, D), jnp.float32),
        jax.ShapeDtypeStruct((K, D), jnp.float32),
    ),
    mesh=_mesh,
    compiler_params=pltpu.CompilerParams(needs_layout_passes=False),
    scratch_types=[
        pltpu.VMEM((2 * CHUNK,), jnp.int32),
        pltpu.VMEM((2 * CHUNK,), jnp.int32),
        pltpu.VMEM((2 * XSTREAMS, 128), jnp.int32),
        pltpu.VMEM((2 * XSTREAMS, 128), jnp.int32),
        pltpu.VMEM((2 * CHUNK, 2 * D), jnp.float32),
        pltpu.VMEM((2 * XPC, 2 * D), jnp.float32),
        pltpu.VMEM((XPC,), jnp.float32),
        pltpu.VMEM((CHUNK, D), jnp.float32),
        pltpu.VMEM((16,), jnp.int32),
        pltpu.VMEM((16,), jnp.int32),
        pltpu.VMEM((16, 2 * D), jnp.float32),
        pltpu.VMEM((K, D), jnp.float32),
        pltpu.SemaphoreType.DMA,
        pltpu.SemaphoreType.DMA,
    ],
)
def _sc_scores(cen_hbm, ctx_hbm, tab2_hbm, neg_hbm,
               pos_hbm, cemb_hbm, nemb_hbm,
               cidx_v, clin_v, xidx_v, xlin_v, crows_v, xrows_v, pos_v,
               cext_v, nidx_v, nlin_v, nrows_v, next_v, sem0, sem1):
    wid = lax.axis_index("s") * NC + lax.axis_index("c")
    iota = lax.iota(jnp.int32, 16)

    @pl.when(wid == 0)
    def _():
        pltpu.sync_copy(neg_hbm, nidx_v.at[pl.ds(0, K)])
        raw = nidx_v[...]
        nlin_v[...] = jnp.where(iota < K, raw >> 1, 0)
        pltpu.async_copy(tab2_hbm.at[nlin_v], nrows_v, sem0).wait()
        for kk in range(K):
            krow = jnp.full((16,), kk, jnp.int32)
            odd = (plsc.load_gather(nidx_v, [krow]) & 1) * D
            for ds_ in range(D // 16):
                v = plsc.load_gather(nrows_v, [krow, odd + ds_ * 16 + iota])
                plsc.store_scatter(next_v, [krow, ds_ * 16 + iota], v)
        pltpu.sync_copy(next_v, nemb_hbm)

    def issue(c, par, sem):
        base = wid * BPW + c * CHUNK
        po_c = par * CHUNK
        po_x = par * XSTREAMS
        pltpu.sync_copy(cen_hbm.at[pl.ds(base, CHUNK)],
                        cidx_v.at[pl.ds(po_c, CHUNK)])
        for j in range(XSTREAMS):
            pltpu.sync_copy(ctx_hbm.at[pl.ds(base * M + j * 128, 128)],
                            xidx_v.at[po_x + j])
        for t in range(CHUNK // 16):
            clin_v[pl.ds(po_c + t * 16, 16)] = (
                cidx_v[pl.ds(po_c + t * 16, 16)] >> 1)
        for j in range(XSTREAMS):
            for t in range(8):
                xlin_v[po_x + j, pl.ds(t * 16, 16)] = (
                    xidx_v[po_x + j, pl.ds(t * 16, 16)] >> 1)
        pltpu.async_copy(tab2_hbm.at[clin_v.at[pl.ds(po_c, CHUNK)]],
                         crows_v.at[pl.ds(po_c, CHUNK)], sem)
        for j in range(XSTREAMS):
            pltpu.async_copy(tab2_hbm.at[xlin_v.at[po_x + j]],
                             xrows_v.at[pl.ds(par * XPC + j * 128, 128)], sem)

    def drain(sem):
        pltpu.make_async_copy(tab2_hbm.at[pl.ds(0, CHUNK)],
                              crows_v.at[pl.ds(0, CHUNK)], sem).wait()
        for j in range(XSTREAMS):
            pltpu.make_async_copy(tab2_hbm.at[pl.ds(0, 128)],
                                  xrows_v.at[pl.ds(0, 128)], sem).wait()

    def compute(c, par):
        base = wid * BPW + c * CHUNK
        po_c = par * CHUNK
        po_x = par * XSTREAMS

        def group_body(g, carry2):
            bidx = g * 16 + iota
            crow = po_c + bidx
            codd = (plsc.load_gather(cidx_v, [crow]) & 1) * D
            rowv = [g * 128 + m + iota * 8 for m in range(M)]
            xrow = []
            xodd = []
            for m in range(M):
                raw = plsc.load_gather(
                    xidx_v, [po_x + (rowv[m] >> 7), rowv[m] & 127])
                xodd.append((raw & 1) * D)
                xrow.append(par * XPC + rowv[m])

            def d_body(dd, accs):
                out = accs
                for du in range(4):
                    dcol = jnp.full((16,), dd * 4 + du, dtype=jnp.int32)
                    cv = plsc.load_gather(crows_v, [crow, codd + dcol])
                    out = tuple(
                        out[m] + cv * plsc.load_gather(
                            xrows_v, [xrow[m], xodd[m] + dcol])
                        for m in range(M)
                    )
                return out

            accs = lax.fori_loop(
                0, D // 4, d_body,
                tuple(jnp.zeros((16,), jnp.float32) for _ in range(M)))
            for m in range(M):
                plsc.store_scatter(pos_v, [rowv[m]], accs[m])

            def b_body(t, carry3):
                brow = jnp.full((16,), g * 16, jnp.int32) + t
                bodd = plsc.load_gather(cidx_v, [po_c + brow]) & 1
                for ds_ in range(D // 16):
                    col = bodd * D + ds_ * 16 + iota
                    v = plsc.load_gather(crows_v, [po_c + brow, col])
                    plsc.store_scatter(cext_v, [brow, ds_ * 16 + iota], v)
                return carry3

            lax.fori_loop(0, 16, b_body, 0)
            return carry2

        lax.fori_loop(0, GROUPS, group_body, 0)
        pltpu.sync_copy(pos_v, pos_hbm.at[pl.ds(base * M, XPC)])
        pltpu.sync_copy(cext_v, cemb_hbm.at[pl.ds(base, CHUNK)])

    issue(0, 0, sem0)

    def pair_body(p, carry):
        c0 = p * 2
        issue(c0 + 1, 1, sem1)
        drain(sem0)
        compute(c0, 0)

        @pl.when(p + 1 < NCHUNKS // 2)
        def _():
            issue(c0 + 2, 0, sem0)

        drain(sem1)
        compute(c0 + 1, 1)
        return carry

    lax.fori_loop(0, NCHUNKS // 2, pair_body, 0)


_BLK = 2048


def _tc_loss(pos_ref, cen_ref, neg_ref, out_ref):
    i = pl.program_id(0)
    pos = pos_ref[...]
    cen = cen_ref[...]
    neg = neg_ref[...]
    negs = lax.dot_general(cen, neg, (((1,), (1,)), ((), ())),
                           preferred_element_type=jnp.float32)
    nmax = jnp.max(negs, axis=1, keepdims=True)
    s = jnp.sum(jnp.exp(negs - nmax), axis=1, keepdims=True)
    a = jnp.maximum(pos, nmax)
    r = a + jnp.log(jnp.exp(pos - a) + jnp.exp(nmax - a) * s) - pos
    part = jnp.sum(r) * (1.0 / (NB * M))

    @pl.when(i == 0)
    def _():
        out_ref[0, 0] = 0.0

    out_ref[0, 0] += part


_tc_call = pl.pallas_call(
    _tc_loss,
    grid=(NB // _BLK,),
    in_specs=[
        pl.BlockSpec((_BLK, M), lambda i: (i, 0)),
        pl.BlockSpec((_BLK, D), lambda i: (i, 0)),
        pl.BlockSpec((K, D), lambda i: (0, 0)),
    ],
    out_specs=pl.BlockSpec(memory_space=pltpu.SMEM),
    out_shape=jax.ShapeDtypeStruct((1, 1), jnp.float32),
)


def kernel(center_words, context_words, embedding, neg_labels):
    cen = center_words.astype(jnp.int32)
    ctx = context_words.astype(jnp.int32).reshape(NB * M)
    tab2 = jnp.concatenate([embedding[0::2], embedding[1::2]], axis=1)
    pos_flat, cemb, nemb = _sc_scores(cen, ctx, tab2,
                                      neg_labels.astype(jnp.int32))
    pos = pos_flat.reshape(NB, M)
    loss = _tc_call(pos, cemb, nemb)
    return loss[0, 0]


# speedup vs baseline: 11.0936x; 11.0936x over previous
import functools

import jax
import jax.numpy as jnp
from jax import lax
from jax.experimental import pallas as pl
from jax.experimental.pallas import tpu as pltpu
from jax.experimental.pallas import tpu_sc as plsc

D = 64
NB = 16384
M = 8
K = 8
NC = 2
NS = 16
NW = NC * NS
BPW = NB // NW
CHUNK = 32
NCHUNKS = BPW // CHUNK
GROUPS = CHUNK // 16
XPC = CHUNK * M
XSTREAMS = XPC // 128

_mesh = plsc.VectorSubcoreMesh(core_axis_name="c", subcore_axis_name="s")


@functools.partial(
    pl.kernel,
    out_type=(
        jax.ShapeDtypeStruct((NB * M,), jnp.float32),
        jax.ShapeDtypeStruct((NB, D), jnp.float32),
        jax.ShapeDtypeStruct((K, D), jnp.float32),
    ),
    mesh=_mesh,
    compiler_params=pltpu.CompilerParams(needs_layout_passes=False),
    scratch_types=[
        pltpu.VMEM((2 * CHUNK,), jnp.int32),
        pltpu.VMEM((2 * CHUNK,), jnp.int32),
        pltpu.VMEM((2 * XSTREAMS, 128), jnp.int32),
        pltpu.VMEM((2 * XSTREAMS, 128), jnp.int32),
        pltpu.VMEM((2 * CHUNK, 2 * D), jnp.float32),
        pltpu.VMEM((2 * XPC, 2 * D), jnp.float32),
        pltpu.VMEM((XPC,), jnp.float32),
        pltpu.VMEM((CHUNK, D), jnp.float32),
        pltpu.VMEM((16,), jnp.int32),
        pltpu.VMEM((16,), jnp.int32),
        pltpu.VMEM((16, 2 * D), jnp.float32),
        pltpu.VMEM((K, D), jnp.float32),
        pltpu.SemaphoreType.DMA,
        pltpu.SemaphoreType.DMA,
    ],
)
def _sc_scores(cen_hbm, ctx_hbm, tab2_hbm, neg_hbm,
               pos_hbm, cemb_hbm, nemb_hbm,
               cidx_v, clin_v, xidx_v, xlin_v, crows_v, xrows_v, pos_v,
               cext_v, nidx_v, nlin_v, nrows_v, next_v, sem0, sem1):
    wid = lax.axis_index("s") * NC + lax.axis_index("c")
    iota = lax.iota(jnp.int32, 16)

    @pl.when(wid == 0)
    def _():
        pltpu.sync_copy(neg_hbm, nidx_v.at[pl.ds(0, K)])
        raw = nidx_v[...]
        nlin_v[...] = jnp.where(iota < K, raw >> 1, 0)
        pltpu.async_copy(tab2_hbm.at[nlin_v], nrows_v, sem0).wait()
        for kk in range(K):
            krow = jnp.full((16,), kk, jnp.int32)
            odd = (plsc.load_gather(nidx_v, [krow]) & 1) * D
            for ds_ in range(D // 16):
                v = plsc.load_gather(nrows_v, [krow, odd + ds_ * 16 + iota])
                plsc.store_scatter(next_v, [krow, ds_ * 16 + iota], v)
        pltpu.sync_copy(next_v, nemb_hbm)

    def issue(c, par, sem):
        base = wid * BPW + c * CHUNK
        po_c = par * CHUNK
        po_x = par * XSTREAMS
        pltpu.sync_copy(cen_hbm.at[pl.ds(base, CHUNK)],
                        cidx_v.at[pl.ds(po_c, CHUNK)])
        for j in range(XSTREAMS):
            pltpu.sync_copy(ctx_hbm.at[pl.ds(base * M + j * 128, 128)],
                            xidx_v.at[po_x + j])
        for t in range(CHUNK // 16):
            clin_v[pl.ds(po_c + t * 16, 16)] = (
                cidx_v[pl.ds(po_c + t * 16, 16)] >> 1)
        for j in range(XSTREAMS):
            for t in range(8):
                xlin_v[po_x + j, pl.ds(t * 16, 16)] = (
                    xidx_v[po_x + j, pl.ds(t * 16, 16)] >> 1)
        pltpu.async_copy(tab2_hbm.at[clin_v.at[pl.ds(po_c, CHUNK)]],
                         crows_v.at[pl.ds(po_c, CHUNK)], sem)
        for j in range(XSTREAMS):
            pltpu.async_copy(tab2_hbm.at[xlin_v.at[po_x + j]],
                             xrows_v.at[pl.ds(par * XPC + j * 128, 128)], sem)

    def drain(sem):
        pltpu.make_async_copy(tab2_hbm.at[pl.ds(0, CHUNK)],
                              crows_v.at[pl.ds(0, CHUNK)], sem).wait()
        for j in range(XSTREAMS):
            pltpu.make_async_copy(tab2_hbm.at[pl.ds(0, 128)],
                                  xrows_v.at[pl.ds(0, 128)], sem).wait()

    def compute(c, par):
        base = wid * BPW + c * CHUNK
        po_c = par * CHUNK
        po_x = par * XSTREAMS

        def group_body(g, carry2):
            bidx = g * 16 + iota
            crow = po_c + bidx
            codd = (plsc.load_gather(cidx_v, [crow]) & 1) * D
            rowv = [g * 128 + m + iota * 8 for m in range(M)]
            xrow = []
            xodd = []
            for m in range(M):
                raw = plsc.load_gather(
                    xidx_v, [po_x + (rowv[m] >> 7), rowv[m] & 127])
                xodd.append((raw & 1) * D)
                xrow.append(par * XPC + rowv[m])

            def d_body(dd, accs):
                out = accs
                for du in range(4):
                    dcol = jnp.full((16,), dd * 4 + du, dtype=jnp.int32)
                    cv = plsc.load_gather(crows_v, [crow, codd + dcol])
                    out = tuple(
                        out[m] + cv * plsc.load_gather(
                            xrows_v, [xrow[m], xodd[m] + dcol])
                        for m in range(M)
                    )
                return out

            accs = lax.fori_loop(
                0, D // 4, d_body,
                tuple(jnp.zeros((16,), jnp.float32) for _ in range(M)))
            for m in range(M):
                plsc.store_scatter(pos_v, [rowv[m]], accs[m])

            def b_body(t, carry3):
                brow = jnp.full((16,), g * 16, jnp.int32) + t
                bodd = plsc.load_gather(cidx_v, [po_c + brow]) & 1
                for ds_ in range(D // 16):
                    col = bodd * D + ds_ * 16 + iota
                    v = plsc.load_gather(crows_v, [po_c + brow, col])
                    plsc.store_scatter(cext_v, [brow, ds_ * 16 + iota], v)
                return carry3

            lax.fori_loop(0, 16, b_body, 0)
            return carry2

        lax.fori_loop(0, GROUPS, group_body, 0)
        pltpu.sync_copy(pos_v, pos_hbm.at[pl.ds(base * M, XPC)])
        pltpu.sync_copy(cext_v, cemb_hbm.at[pl.ds(base, CHUNK)])

    issue(0, 0, sem0)

    def pair_body(p, carry):
        c0 = p * 2
        issue(c0 + 1, 1, sem1)
        drain(sem0)
        compute(c0, 0)

        @pl.when(p + 1 < NCHUNKS // 2)
        def _():
            issue(c0 + 2, 0, sem0)

        drain(sem1)
        compute(c0 + 1, 1)
        return carry

    lax.fori_loop(0, NCHUNKS // 2, pair_body, 0)


_BLK = 2048


def _tc_loss(pos_ref, cen_ref, neg_ref, out_ref):
    i = pl.program_id(0)
    pos = pos_ref[...]
    cen = cen_ref[...]
    neg = neg_ref[...]
    negs = lax.dot_general(cen, neg, (((1,), (1,)), ((), ())),
                           preferred_element_type=jnp.float32)
    nmax = jnp.max(negs, axis=1, keepdims=True)
    s = jnp.sum(jnp.exp(negs - nmax), axis=1, keepdims=True)
    a = jnp.maximum(pos, nmax)
    r = a + jnp.log(jnp.exp(pos - a) + jnp.exp(nmax - a) * s) - pos
    part = jnp.sum(r) * (1.0 / (NB * M))

    @pl.when(i == 0)
    def _():
        out_ref[0, 0] = 0.0

    out_ref[0, 0] += part


_tc_call = pl.pallas_call(
    _tc_loss,
    grid=(NB // _BLK,),
    in_specs=[
        pl.BlockSpec((_BLK, M), lambda i: (i, 0)),
        pl.BlockSpec((_BLK, D), lambda i: (i, 0)),
        pl.BlockSpec((K, D), lambda i: (0, 0)),
    ],
    out_specs=pl.BlockSpec(memory_space=pltpu.SMEM),
    out_shape=jax.ShapeDtypeStruct((1, 1), jnp.float32),
)


def kernel(center_words, context_words, embedding, neg_labels):
    cen = center_words.astype(jnp.int32)
    ctx = context_words.astype(jnp.int32).reshape(NB * M)
    tab2 = embedding.reshape(500000, 2 * D)
    pos_flat, cemb, nemb = _sc_scores(cen, ctx, tab2,
                                      neg_labels.astype(jnp.int32))
    pos = pos_flat.reshape(NB, M)
    loss = _tc_call(pos, cemb, nemb)
    return loss[0, 0]
